# Initial kernel scaffold; baseline (speedup 1.0000x reference)
import jax
import jax.numpy as jnp
from jax.experimental import pallas as pl


def kernel(x, pairs, pair_blocks):
    raise NotImplementedError("write your pallas kernel here")


# speedup vs baseline: 1.5296x; 1.5296x over previous
import functools

import jax
import jax.numpy as jnp
from jax import lax
from jax.experimental import pallas as pl
from jax.experimental.pallas import tpu as pltpu
from jax.experimental.pallas import tpu_sc as plsc

_LANES = 16


def _column_map(pairs, pair_blocks, d_model):
    n = pairs.shape[0]
    idx0 = pairs[:, 0].astype(jnp.int32)
    idx1 = pairs[:, 1].astype(jnp.int32)
    t = pair_blocks.astype(jnp.float32)
    t00 = t[:, 0, 0] + 1.0
    t01 = t[:, 0, 1]
    t10 = t[:, 1, 0]
    t11 = t[:, 1, 1] + 1.0
    ar = jnp.arange(n, dtype=jnp.int32)
    winner = jnp.full((d_model,), -1, jnp.int32)
    winner = winner.at[idx0].set(ar)
    winner = winner.at[idx1].set(ar + n)
    valid = winner >= 0
    w = jnp.where(valid, winner, 0)
    slot = w // n
    p = w % n
    zero_f = jnp.float32(0.0)
    a = jnp.where(valid, jnp.where(slot == 0, t00[p], t01[p]), zero_f)
    b = jnp.where(valid, jnp.where(slot == 0, t10[p], t11[p]), zero_f)
    g0 = jnp.where(valid, idx0[p], 0)
    g1 = jnp.where(valid, idx1[p], 0)
    return g0, g1, a, b


def _sc_apply(x2d, g0, g1, a, b):
    rows, d = x2d.shape
    info = plsc.get_sparse_core_info()
    nc, ns = info.num_cores, info.num_subcores
    nw = nc * ns
    rows_per_w = rows // nw
    k_rows = 8
    chunks = rows_per_w // k_rows
    groups = d // _LANES
    mesh = plsc.VectorSubcoreMesh(core_axis_name="c", subcore_axis_name="s")

    @functools.partial(
        pl.kernel,
        mesh=mesh,
        compiler_params=pltpu.CompilerParams(needs_layout_passes=False),
        out_type=jax.ShapeDtypeStruct((rows * d,), jnp.float32),
        scratch_types=[
            pltpu.VMEM((d,), jnp.int32),
            pltpu.VMEM((d,), jnp.int32),
            pltpu.VMEM((d,), jnp.float32),
            pltpu.VMEM((d,), jnp.float32),
            pltpu.VMEM((k_rows * d,), jnp.float32),
            pltpu.VMEM((k_rows * d,), jnp.float32),
        ],
    )
    def run(x_hbm, g0_hbm, g1_hbm, a_hbm, b_hbm, y_hbm,
            g0_v, g1_v, a_v, b_v, xbuf, ybuf):
        wid = lax.axis_index("s") * nc + lax.axis_index("c")
        base = wid * rows_per_w * d
        pltpu.sync_copy(g0_hbm, g0_v)
        pltpu.sync_copy(g1_hbm, g1_v)
        pltpu.sync_copy(a_hbm, a_v)
        pltpu.sync_copy(b_hbm, b_v)

        def chunk_body(ci, carry):
            off0 = base + ci * (k_rows * d)
            pltpu.sync_copy(x_hbm.at[pl.ds(off0, k_rows * d)], xbuf)

            def col_body(g, c2):
                off = pl.multiple_of(g * _LANES, _LANES)
                i0 = g0_v[pl.ds(off, _LANES)]
                i1 = g1_v[pl.ds(off, _LANES)]
                av = a_v[pl.ds(off, _LANES)]
                bv = b_v[pl.ds(off, _LANES)]
                for kk in range(k_rows):
                    rbase = jnp.full((_LANES,), kk * d, jnp.int32)
                    xg0 = plsc.load_gather(xbuf, [rbase + i0])
                    xg1 = plsc.load_gather(xbuf, [rbase + i1])
                    ybuf[pl.ds(off + kk * d, _LANES)] = xg0 * av + xg1 * bv
                return c2

            lax.fori_loop(0, groups, col_body, 0)
            pltpu.sync_copy(ybuf, y_hbm.at[pl.ds(off0, k_rows * d)])
            return carry

        lax.fori_loop(0, chunks, chunk_body, 0)

    return run(x2d.reshape(-1), g0, g1, a, b)


def kernel(x, pairs, pair_blocks):
    batch, seq, d = x.shape
    g0, g1, a, b = _column_map(pairs, pair_blocks, d)
    x2d = x.reshape(batch * seq, d)
    y_flat = _sc_apply(x2d, g0, g1, a, b)
    return y_flat.reshape(batch, seq, d)


# speedup vs baseline: 1.6929x; 1.1068x over previous
import functools

import jax
import jax.numpy as jnp
from jax import lax
from jax.experimental import pallas as pl
from jax.experimental.pallas import tpu as pltpu
from jax.experimental.pallas import tpu_sc as plsc

_LANES = 16


def _sc_run(x_flat, pairs_flat, pb_flat, rows, d, n_pairs):
    info = plsc.get_sparse_core_info()
    nc, ns = info.num_cores, info.num_subcores
    nw = nc * ns
    rows_per_w = rows // nw
    k_rows = 8
    chunks = rows_per_w // k_rows
    groups = d // _LANES
    mesh = plsc.VectorSubcoreMesh(core_axis_name="c", subcore_axis_name="s")

    @functools.partial(
        pl.kernel,
        mesh=mesh,
        compiler_params=pltpu.CompilerParams(needs_layout_passes=False),
        out_type=jax.ShapeDtypeStruct((rows * d,), jnp.float32),
        scratch_types=[
            pltpu.VMEM((2 * n_pairs,), jnp.int32),
            pltpu.VMEM((4 * n_pairs,), jnp.float32),
            pltpu.VMEM((d,), jnp.int32),
            pltpu.VMEM((d,), jnp.int32),
            pltpu.VMEM((d,), jnp.int32),
            pltpu.VMEM((d,), jnp.float32),
            pltpu.VMEM((d,), jnp.float32),
            pltpu.VMEM((k_rows * d,), jnp.float32),
            pltpu.VMEM((k_rows * d,), jnp.float32),
        ],
    )
    def run(x_hbm, keys_hbm, pb_hbm, y_hbm,
            keys_v, pb_v, win_v, g0_v, g1_v, a_v, b_v, xbuf, ybuf):
        wid = lax.axis_index("s") * nc + lax.axis_index("c")
        base = wid * rows_per_w * d
        pltpu.sync_copy(keys_hbm, keys_v)
        pltpu.sync_copy(pb_hbm, pb_v)

        neg1 = jnp.full((_LANES,), -1, jnp.int32)
        lane_ids = jnp.arange(_LANES, dtype=jnp.int32)
        lane_masks = [lane_ids == l for l in range(_LANES)]

        def init_body(g, c):
            win_v[pl.ds(pl.multiple_of(g * _LANES, _LANES), _LANES)] = neg1
            return c

        lax.fori_loop(0, groups, init_body, 0)

        key_groups = (2 * n_pairs) // _LANES

        def scat_body(g, c):
            off = pl.multiple_of(g * _LANES, _LANES)
            kvec = keys_v[pl.ds(off, _LANES)]
            vals = jnp.full((_LANES,), 1, jnp.int32) * off + lane_ids
            for l in range(_LANES):
                plsc.store_scatter(win_v, [kvec], vals, mask=lane_masks[l])
            return c

        lax.fori_loop(0, key_groups, scat_body, 0)

        one_f = jnp.full((_LANES,), 1.0, jnp.float32)
        zero_f = jnp.zeros((_LANES,), jnp.float32)
        zero_i = jnp.zeros((_LANES,), jnp.int32)

        def derive_body(g, c):
            off = pl.multiple_of(g * _LANES, _LANES)
            w = win_v[pl.ds(off, _LANES)]
            valid = w >= 0
            wv = jnp.where(valid, w, 0)
            slot = wv // n_pairs
            p = wv - slot * n_pairs
            i0 = plsc.load_gather(keys_v, [p])
            i1 = plsc.load_gather(keys_v, [p + n_pairs])
            av = plsc.load_gather(pb_v, [4 * p + slot])
            bv = plsc.load_gather(pb_v, [4 * p + 2 + slot])
            slot_is0 = slot == 0
            av = av + jnp.where(slot_is0, one_f, zero_f)
            bv = bv + jnp.where(slot_is0, zero_f, one_f)
            g0_v[pl.ds(off, _LANES)] = jnp.where(valid, i0, zero_i)
            g1_v[pl.ds(off, _LANES)] = jnp.where(valid, i1, zero_i)
            a_v[pl.ds(off, _LANES)] = jnp.where(valid, av, zero_f)
            b_v[pl.ds(off, _LANES)] = jnp.where(valid, bv, zero_f)
            return c

        lax.fori_loop(0, groups, derive_body, 0)

        def chunk_body(ci, carry):
            off0 = base + ci * (k_rows * d)
            pltpu.sync_copy(x_hbm.at[pl.ds(off0, k_rows * d)], xbuf)

            def col_body(g, c2):
                off = pl.multiple_of(g * _LANES, _LANES)
                i0 = g0_v[pl.ds(off, _LANES)]
                i1 = g1_v[pl.ds(off, _LANES)]
                av = a_v[pl.ds(off, _LANES)]
                bv = b_v[pl.ds(off, _LANES)]
                for kk in range(k_rows):
                    rbase = jnp.full((_LANES,), kk * d, jnp.int32)
                    xg0 = plsc.load_gather(xbuf, [rbase + i0])
                    xg1 = plsc.load_gather(xbuf, [rbase + i1])
                    ybuf[pl.ds(off + kk * d, _LANES)] = xg0 * av + xg1 * bv
                return c2

            lax.fori_loop(0, groups, col_body, 0)
            pltpu.sync_copy(ybuf, y_hbm.at[pl.ds(off0, k_rows * d)])
            return carry

        lax.fori_loop(0, chunks, chunk_body, 0)

    return run(x_flat, pairs_flat, pb_flat)


def kernel(x, pairs, pair_blocks):
    batch, seq, d = x.shape
    n_pairs = pairs.shape[0]
    x_flat = x.reshape(-1)
    keys = pairs.astype(jnp.int32).T.reshape(-1)
    pb_flat = pair_blocks.astype(jnp.float32).reshape(-1)
    y_flat = _sc_run(x_flat, keys, pb_flat, batch * seq, d, n_pairs)
    return y_flat.reshape(batch, seq, d)


# speedup vs baseline: 2.1473x; 1.2684x over previous
import functools

import jax
import jax.numpy as jnp
from jax import lax
from jax.experimental import pallas as pl
from jax.experimental.pallas import tpu as pltpu
from jax.experimental.pallas import tpu_sc as plsc

_LANES = 16


def _sc_run(x_flat, pairs_flat, pb_flat, rows, d, n_pairs):
    info = plsc.get_sparse_core_info()
    nc, ns = info.num_cores, info.num_subcores
    nw = nc * ns
    rows_per_w = rows // nw
    k_rows = 8
    chunks = rows_per_w // k_rows
    groups = d // _LANES
    mesh = plsc.VectorSubcoreMesh(core_axis_name="c", subcore_axis_name="s")

    @functools.partial(
        pl.kernel,
        mesh=mesh,
        compiler_params=pltpu.CompilerParams(needs_layout_passes=False),
        out_type=jax.ShapeDtypeStruct((rows, d), jnp.float32),
        scratch_types=[
            pltpu.VMEM((2 * n_pairs,), jnp.int32),
            pltpu.VMEM((4 * n_pairs,), jnp.float32),
            pltpu.VMEM((d,), jnp.int32),
            pltpu.VMEM((d,), jnp.int32),
            pltpu.VMEM((d,), jnp.int32),
            pltpu.VMEM((d,), jnp.float32),
            pltpu.VMEM((d,), jnp.float32),
            pltpu.VMEM((k_rows, d), jnp.float32),
            pltpu.VMEM((k_rows, d), jnp.float32),
        ],
    )
    def run(x_hbm, keys_hbm, pb_hbm, y_hbm,
            keys_v, pb_v, win_v, g0_v, g1_v, a_v, b_v, xbuf, ybuf):
        wid = lax.axis_index("s") * nc + lax.axis_index("c")
        base = wid * rows_per_w
        pltpu.sync_copy(keys_hbm, keys_v)
        pltpu.sync_copy(pb_hbm, pb_v)

        neg1 = jnp.full((_LANES,), -1, jnp.int32)
        lane_ids = jnp.arange(_LANES, dtype=jnp.int32)
        lane_masks = [lane_ids == l for l in range(_LANES)]

        def init_body(g, c):
            win_v[pl.ds(pl.multiple_of(g * _LANES, _LANES), _LANES)] = neg1
            return c

        lax.fori_loop(0, groups, init_body, 0)

        key_groups = (2 * n_pairs) // _LANES

        def scat_body(g, c):
            off = pl.multiple_of(g * _LANES, _LANES)
            kvec = keys_v[pl.ds(off, _LANES)]
            vals = jnp.full((_LANES,), 1, jnp.int32) * off + lane_ids
            for l in range(_LANES):
                plsc.store_scatter(win_v, [kvec], vals, mask=lane_masks[l])
            return c

        lax.fori_loop(0, key_groups, scat_body, 0)

        one_f = jnp.full((_LANES,), 1.0, jnp.float32)
        zero_f = jnp.zeros((_LANES,), jnp.float32)
        zero_i = jnp.zeros((_LANES,), jnp.int32)

        def derive_body(g, c):
            off = pl.multiple_of(g * _LANES, _LANES)
            w = win_v[pl.ds(off, _LANES)]
            valid = w >= 0
            wv = jnp.where(valid, w, 0)
            slot = wv // n_pairs
            p = wv - slot * n_pairs
            i0 = plsc.load_gather(keys_v, [p])
            i1 = plsc.load_gather(keys_v, [p + n_pairs])
            av = plsc.load_gather(pb_v, [4 * p + slot])
            bv = plsc.load_gather(pb_v, [4 * p + 2 + slot])
            slot_is0 = slot == 0
            av = av + jnp.where(slot_is0, one_f, zero_f)
            bv = bv + jnp.where(slot_is0, zero_f, one_f)
            g0_v[pl.ds(off, _LANES)] = jnp.where(valid, i0, zero_i)
            g1_v[pl.ds(off, _LANES)] = jnp.where(valid, i1, zero_i)
            a_v[pl.ds(off, _LANES)] = jnp.where(valid, av, zero_f)
            b_v[pl.ds(off, _LANES)] = jnp.where(valid, bv, zero_f)
            return c

        lax.fori_loop(0, groups, derive_body, 0)

        def chunk_body(ci, carry):
            row0 = base + ci * k_rows
            pltpu.sync_copy(x_hbm.at[pl.ds(row0, k_rows)], xbuf)

            def col_body(g, c2):
                off = pl.multiple_of(g * _LANES, _LANES)
                i0 = g0_v[pl.ds(off, _LANES)]
                i1 = g1_v[pl.ds(off, _LANES)]
                av = a_v[pl.ds(off, _LANES)]
                bv = b_v[pl.ds(off, _LANES)]
                for kk in range(k_rows):
                    rowv = jnp.full((_LANES,), kk, jnp.int32)
                    xg0 = plsc.load_gather(xbuf, [rowv, i0])
                    xg1 = plsc.load_gather(xbuf, [rowv, i1])
                    ybuf[kk, pl.ds(off, _LANES)] = xg0 * av + xg1 * bv
                return c2

            lax.fori_loop(0, groups, col_body, 0)
            pltpu.sync_copy(ybuf, y_hbm.at[pl.ds(row0, k_rows)])
            return carry

        lax.fori_loop(0, chunks, chunk_body, 0)

    return run(x_flat, pairs_flat, pb_flat)


def kernel(x, pairs, pair_blocks):
    batch, seq, d = x.shape
    n_pairs = pairs.shape[0]
    x2d = x.reshape(batch * seq, d)
    keys = pairs.astype(jnp.int32).T.reshape(-1)
    pb_flat = pair_blocks.astype(jnp.float32).reshape(-1)
    y2d = _sc_run(x2d, keys, pb_flat, batch * seq, d, n_pairs)
    return y2d.reshape(batch, seq, d)


# speedup vs baseline: 2.5164x; 1.1719x over previous
import functools

import jax
import jax.numpy as jnp
from jax import lax
from jax.experimental import pallas as pl
from jax.experimental.pallas import tpu as pltpu
from jax.experimental.pallas import tpu_sc as plsc

_LANES = 16


def _sc_run(x2d, keys, pb_flat, rows, d, n_pairs):
    info = plsc.get_sparse_core_info()
    nc, ns = info.num_cores, info.num_subcores
    nw = nc * ns
    rows_per_w = rows // nw
    k_rows = 4
    chunks = rows_per_w // k_rows
    groups = d // _LANES
    mesh = plsc.VectorSubcoreMesh(core_axis_name="c", subcore_axis_name="s")

    @functools.partial(
        pl.kernel,
        mesh=mesh,
        compiler_params=pltpu.CompilerParams(needs_layout_passes=False),
        out_type=jax.ShapeDtypeStruct((rows, d), jnp.float32),
        scratch_types=[
            pltpu.VMEM((2 * n_pairs,), jnp.int32),
            pltpu.VMEM((4 * n_pairs,), jnp.float32),
            pltpu.VMEM((d,), jnp.int32),
            pltpu.VMEM((d,), jnp.float32),
            pltpu.VMEM((d,), jnp.float32),
            pltpu.VMEM((d,), jnp.int32),
            pltpu.VMEM((k_rows, d), jnp.float32),
            pltpu.VMEM((k_rows, d), jnp.float32),
            pltpu.VMEM((k_rows, d), jnp.float32),
            pltpu.VMEM((k_rows, d), jnp.float32),
            pltpu.SemaphoreType.DMA,
            pltpu.SemaphoreType.DMA,
            pltpu.SemaphoreType.DMA,
            pltpu.SemaphoreType.DMA,
        ],
    )
    def run(x_hbm, keys_hbm, pb_hbm, y_hbm,
            keys_v, pb_v, win_v, dc_v, oc_v, go_v,
            xb0, xb1, yb0, yb1, isem0, isem1, osem0, osem1):
        wid = lax.axis_index("s") * nc + lax.axis_index("c")
        base = wid * rows_per_w
        pltpu.sync_copy(keys_hbm, keys_v)
        pltpu.sync_copy(pb_hbm, pb_v)

        neg1 = jnp.full((_LANES,), -1, jnp.int32)
        lane_ids = jnp.arange(_LANES, dtype=jnp.int32)
        lane_masks = [lane_ids == l for l in range(_LANES)]

        def init_body(g, c):
            win_v[pl.ds(pl.multiple_of(g * _LANES, _LANES), _LANES)] = neg1
            return c

        lax.fori_loop(0, groups, init_body, 0)

        key_groups = (2 * n_pairs) // _LANES

        def scat_body(g, c):
            off = pl.multiple_of(g * _LANES, _LANES)
            kvec = keys_v[pl.ds(off, _LANES)]
            vals = jnp.full((_LANES,), 1, jnp.int32) * off + lane_ids
            for l in range(_LANES):
                plsc.store_scatter(win_v, [kvec], vals, mask=lane_masks[l])
            return c

        lax.fori_loop(0, key_groups, scat_body, 0)

        one_f = jnp.full((_LANES,), 1.0, jnp.float32)
        zero_f = jnp.zeros((_LANES,), jnp.float32)
        zero_i = jnp.zeros((_LANES,), jnp.int32)

        def derive_body(g, c):
            off = pl.multiple_of(g * _LANES, _LANES)
            w = win_v[pl.ds(off, _LANES)]
            valid = w >= 0
            wv = jnp.where(valid, w, 0)
            slot = wv // n_pairs
            p = wv - slot * n_pairs
            dc = plsc.load_gather(pb_v, [4 * p + 3 * slot]) + one_f
            oc = plsc.load_gather(pb_v, [4 * p + 2 - slot])
            go = plsc.load_gather(keys_v, [p + n_pairs - n_pairs * slot])
            dc_v[pl.ds(off, _LANES)] = jnp.where(valid, dc, zero_f)
            oc_v[pl.ds(off, _LANES)] = jnp.where(valid, oc, zero_f)
            go_v[pl.ds(off, _LANES)] = jnp.where(valid, go, zero_i)
            return c

        lax.fori_loop(0, groups, derive_body, 0)

        def in_slice(ci):
            return x_hbm.at[pl.ds(base + ci * k_rows, k_rows)]

        def out_slice(ci):
            return y_hbm.at[pl.ds(base + ci * k_rows, k_rows)]

        def compute(xbuf, ybuf):
            def col_body(g, c2):
                off = pl.multiple_of(g * _LANES, _LANES)
                dcv = dc_v[pl.ds(off, _LANES)]
                ocv = oc_v[pl.ds(off, _LANES)]
                gov = go_v[pl.ds(off, _LANES)]
                for kk in range(k_rows):
                    rowv = jnp.full((_LANES,), kk, jnp.int32)
                    xl = xbuf[kk, pl.ds(off, _LANES)]
                    xg = plsc.load_gather(xbuf, [rowv, gov])
                    ybuf[kk, pl.ds(off, _LANES)] = xl * dcv + xg * ocv
                return c2

            lax.fori_loop(0, groups, col_body, 0)

        pltpu.async_copy(in_slice(0), xb0, isem0)

        def pair_body(i, carry):
            ci = 2 * i
            pltpu.async_copy(in_slice(ci + 1), xb1, isem1)
            pltpu.make_async_copy(in_slice(ci), xb0, isem0).wait()

            @pl.when(i >= 1)
            def _():
                pltpu.make_async_copy(yb0, out_slice(ci - 2), osem0).wait()

            compute(xb0, yb0)
            pltpu.async_copy(yb0, out_slice(ci), osem0)

            @pl.when(ci + 2 < chunks)
            def _():
                pltpu.async_copy(in_slice(ci + 2), xb0, isem0)

            pltpu.make_async_copy(in_slice(ci + 1), xb1, isem1).wait()

            @pl.when(i >= 1)
            def _():
                pltpu.make_async_copy(yb1, out_slice(ci - 1), osem1).wait()

            compute(xb1, yb1)
            pltpu.async_copy(yb1, out_slice(ci + 1), osem1)
            return carry

        lax.fori_loop(0, chunks // 2, pair_body, 0)
        pltpu.make_async_copy(yb0, out_slice(chunks - 2), osem0).wait()
        pltpu.make_async_copy(yb1, out_slice(chunks - 1), osem1).wait()

    return run(x2d, keys, pb_flat)


def kernel(x, pairs, pair_blocks):
    batch, seq, d = x.shape
    n_pairs = pairs.shape[0]
    x2d = x.reshape(batch * seq, d)
    keys = pairs.astype(jnp.int32).T.reshape(-1)
    pb_flat = pair_blocks.astype(jnp.float32).reshape(-1)
    y2d = _sc_run(x2d, keys, pb_flat, batch * seq, d, n_pairs)
    return y2d.reshape(batch, seq, d)


# speedup vs baseline: 7.7791x; 3.0913x over previous
import functools

import jax
import jax.numpy as jnp
from jax import lax
from jax.experimental import pallas as pl
from jax.experimental.pallas import tpu as pltpu
from jax.experimental.pallas import tpu_sc as plsc

_LANES = 16


def _sc_run(x2d, keys, pb_flat, rows, d, n_pairs):
    info = plsc.get_sparse_core_info()
    nc, ns = info.num_cores, info.num_subcores
    nw = nc * ns
    rows_per_w = rows // nw
    k_rows = 4
    chunks = rows_per_w // k_rows
    groups = d // _LANES
    mesh = plsc.VectorSubcoreMesh(core_axis_name="c", subcore_axis_name="s")

    @functools.partial(
        pl.kernel,
        mesh=mesh,
        compiler_params=pltpu.CompilerParams(needs_layout_passes=False),
        out_type=jax.ShapeDtypeStruct((rows, d), jnp.float32),
        scratch_types=[
            pltpu.VMEM((2 * n_pairs,), jnp.int32),
            pltpu.VMEM((4 * n_pairs,), jnp.float32),
            pltpu.VMEM((d,), jnp.int32),
            pltpu.VMEM((d,), jnp.float32),
            pltpu.VMEM((d,), jnp.float32),
            pltpu.VMEM((d,), jnp.int32),
            pltpu.VMEM((k_rows, d), jnp.float32),
            pltpu.VMEM((k_rows, d), jnp.float32),
            pltpu.VMEM((k_rows, d), jnp.float32),
            pltpu.VMEM((k_rows, d), jnp.float32),
            pltpu.SemaphoreType.DMA,
            pltpu.SemaphoreType.DMA,
            pltpu.SemaphoreType.DMA,
            pltpu.SemaphoreType.DMA,
        ],
    )
    def run(x_hbm, keys_hbm, pb_hbm, y_hbm,
            keys_v, pb_v, win_v, dc_v, oc_v, go_v,
            xb0, xb1, yb0, yb1, isem0, isem1, osem0, osem1):
        wid = lax.axis_index("s") * nc + lax.axis_index("c")
        base = wid * rows_per_w
        pltpu.sync_copy(keys_hbm, keys_v)
        pltpu.sync_copy(pb_hbm, pb_v)

        neg1 = jnp.full((_LANES,), -1, jnp.int32)
        lane_ids = jnp.arange(_LANES, dtype=jnp.int32)
        lane_masks = [lane_ids == l for l in range(_LANES)]

        @plsc.parallel_loop(0, groups, unroll=4)
        def init_body(g):
            win_v[pl.ds(pl.multiple_of(g * _LANES, _LANES), _LANES)] = neg1

        key_groups = (2 * n_pairs) // _LANES

        def scat_body(g, c):
            off = pl.multiple_of(g * _LANES, _LANES)
            kvec = keys_v[pl.ds(off, _LANES)]
            vals = jnp.full((_LANES,), 1, jnp.int32) * off + lane_ids
            for l in range(_LANES):
                plsc.store_scatter(win_v, [kvec], vals, mask=lane_masks[l])
            return c

        lax.fori_loop(0, key_groups, scat_body, 0)

        one_f = jnp.full((_LANES,), 1.0, jnp.float32)
        zero_f = jnp.zeros((_LANES,), jnp.float32)
        zero_i = jnp.zeros((_LANES,), jnp.int32)

        @plsc.parallel_loop(0, groups, unroll=2)
        def derive_body(g):
            off = pl.multiple_of(g * _LANES, _LANES)
            w = win_v[pl.ds(off, _LANES)]
            valid = w >= 0
            wv = jnp.where(valid, w, 0)
            slot = wv // n_pairs
            p = wv - slot * n_pairs
            dc = plsc.load_gather(pb_v, [4 * p + 3 * slot]) + one_f
            oc = plsc.load_gather(pb_v, [4 * p + 2 - slot])
            go = plsc.load_gather(keys_v, [p + n_pairs - n_pairs * slot])
            dc_v[pl.ds(off, _LANES)] = jnp.where(valid, dc, zero_f)
            oc_v[pl.ds(off, _LANES)] = jnp.where(valid, oc, zero_f)
            go_v[pl.ds(off, _LANES)] = jnp.where(valid, go, zero_i)

        def in_slice(ci):
            return x_hbm.at[pl.ds(base + ci * k_rows, k_rows)]

        def out_slice(ci):
            return y_hbm.at[pl.ds(base + ci * k_rows, k_rows)]

        def compute(xbuf, ybuf):
            @plsc.parallel_loop(0, groups, unroll=4)
            def col_body(g):
                off = pl.multiple_of(g * _LANES, _LANES)
                dcv = dc_v[pl.ds(off, _LANES)]
                ocv = oc_v[pl.ds(off, _LANES)]
                gov = go_v[pl.ds(off, _LANES)]
                for kk in range(k_rows):
                    rowv = jnp.full((_LANES,), kk, jnp.int32)
                    xl = xbuf[kk, pl.ds(off, _LANES)]
                    xg = plsc.load_gather(xbuf, [rowv, gov])
                    ybuf[kk, pl.ds(off, _LANES)] = xl * dcv + xg * ocv

        pltpu.async_copy(in_slice(0), xb0, isem0)

        def pair_body(i, carry):
            ci = 2 * i
            pltpu.async_copy(in_slice(ci + 1), xb1, isem1)
            pltpu.make_async_copy(in_slice(ci), xb0, isem0).wait()

            @pl.when(i >= 1)
            def _():
                pltpu.make_async_copy(yb0, out_slice(ci - 2), osem0).wait()

            compute(xb0, yb0)
            pltpu.async_copy(yb0, out_slice(ci), osem0)

            @pl.when(ci + 2 < chunks)
            def _():
                pltpu.async_copy(in_slice(ci + 2), xb0, isem0)

            pltpu.make_async_copy(in_slice(ci + 1), xb1, isem1).wait()

            @pl.when(i >= 1)
            def _():
                pltpu.make_async_copy(yb1, out_slice(ci - 1), osem1).wait()

            compute(xb1, yb1)
            pltpu.async_copy(yb1, out_slice(ci + 1), osem1)
            return carry

        lax.fori_loop(0, chunks // 2, pair_body, 0)
        pltpu.make_async_copy(yb0, out_slice(chunks - 2), osem0).wait()
        pltpu.make_async_copy(yb1, out_slice(chunks - 1), osem1).wait()

    return run(x2d, keys, pb_flat)


def kernel(x, pairs, pair_blocks):
    batch, seq, d = x.shape
    n_pairs = pairs.shape[0]
    x2d = x.reshape(batch * seq, d)
    keys = pairs.astype(jnp.int32).T.reshape(-1)
    pb_flat = pair_blocks.astype(jnp.float32).reshape(-1)
    y2d = _sc_run(x2d, keys, pb_flat, batch * seq, d, n_pairs)
    return y2d.reshape(batch, seq, d)
